# Initial kernel scaffold; baseline (speedup 1.0000x reference)
import jax
import jax.numpy as jnp
from jax.experimental import pallas as pl


def kernel(X_c):
    raise NotImplementedError("write your pallas kernel here")


# speedup vs baseline: 10.3566x; 10.3566x over previous
import jax
import jax.numpy as jnp
from jax.experimental import pallas as pl
from jax.experimental.pallas import tpu as pltpu

N = 4096
D = 256
K = 16
INV_T = 10.0
BLOCK = 256
NEG = -3.0


def _body(xb_ref, xf_ref, o_ref):
    xb = xb_ref[...]
    xf = xf_ref[...]
    nb = jnp.maximum(jnp.sqrt(jnp.sum(xb * xb, axis=-1, keepdims=True)), 1e-12)
    nf = jnp.maximum(jnp.sqrt(jnp.sum(xf * xf, axis=-1, keepdims=True)), 1e-12)
    xb = xb / nb
    xf = xf / nf
    sim = jax.lax.dot_general(
        xb, xf, (((1,), (1,)), ((), ())), preferred_element_type=jnp.float32
    )
    w = sim
    m0 = jnp.max(w, axis=-1, keepdims=True)
    t = m0
    for j in range(K - 1):
        w = jnp.where(w >= t, NEG, w)
        t = jnp.max(w, axis=-1, keepdims=True)
    e = jnp.where(sim >= t, jnp.exp((sim - m0) * INV_T), 0.0)
    s = jnp.sum(e, axis=-1, keepdims=True)
    o_ref[...] = e / s


def kernel(X_c):
    return pl.pallas_call(
        _body,
        grid=(N // BLOCK,),
        in_specs=[
            pl.BlockSpec((BLOCK, D), lambda i: (i, 0)),
            pl.BlockSpec((N, D), lambda i: (0, 0)),
        ],
        out_specs=pl.BlockSpec((BLOCK, N), lambda i: (i, 0)),
        out_shape=jax.ShapeDtypeStruct((N, N), jnp.float32),
        compiler_params=pltpu.CompilerParams(
            dimension_semantics=("arbitrary",),
        ),
    )(X_c, X_c)


# speedup vs baseline: 12.0960x; 1.1679x over previous
import jax
import jax.numpy as jnp
from jax.experimental import pallas as pl
from jax.experimental.pallas import tpu as pltpu

N = 4096
D = 256
K = 16
INV_T = 10.0
BLOCK = 256
NEG = -3.0


def _norm_body(x_ref, o_ref):
    x = x_ref[...]
    n = jnp.maximum(jnp.sqrt(jnp.sum(x * x, axis=-1, keepdims=True)), 1e-12)
    o_ref[...] = x / n


def _threshold_full(sim, m0):
    w = sim
    t = m0
    for _ in range(K - 1):
        w = jnp.where(w >= t, NEG, w)
        t = jnp.max(w, axis=-1, keepdims=True)
    return t


def _body(xb_ref, xf_ref, o_ref):
    xb = xb_ref[...]
    xf = xf_ref[...]
    sim = jax.lax.dot_general(
        xb, xf, (((1,), (1,)), ((), ())), preferred_element_type=jnp.float32
    )

    r3 = sim.reshape(BLOCK, N // 128, 128)
    t1 = jnp.max(r3, axis=1)
    w3 = jnp.where(r3 == t1[:, None, :], NEG, r3)
    t2 = jnp.max(w3, axis=1)
    w3 = jnp.where(w3 == t2[:, None, :], NEG, w3)
    t3 = jnp.max(w3, axis=1)
    w3 = jnp.where(w3 == t3[:, None, :], NEG, w3)
    t4 = jnp.max(w3, axis=1)
    cand = jnp.concatenate([t1, t2, t3, t4], axis=-1)

    m0 = jnp.max(t1, axis=-1, keepdims=True)
    t = _threshold_full(cand, m0)

    count = jnp.sum((sim >= t).astype(jnp.float32), axis=-1, keepdims=True)
    bad = count > float(K)
    t = jax.lax.cond(
        jnp.any(bad),
        lambda: jnp.where(bad, _threshold_full(sim, m0), t),
        lambda: t,
    )

    e = jnp.where(sim >= t, jnp.exp((sim - m0) * INV_T), 0.0)
    s = jnp.sum(e, axis=-1, keepdims=True)
    o_ref[...] = e / s


def kernel(X_c):
    Xn = pl.pallas_call(
        _norm_body,
        grid=(4,),
        in_specs=[pl.BlockSpec((N // 4, D), lambda i: (i, 0))],
        out_specs=pl.BlockSpec((N // 4, D), lambda i: (i, 0)),
        out_shape=jax.ShapeDtypeStruct((N, D), jnp.float32),
    )(X_c)
    return pl.pallas_call(
        _body,
        grid=(N // BLOCK,),
        in_specs=[
            pl.BlockSpec((BLOCK, D), lambda i: (i, 0)),
            pl.BlockSpec((N, D), lambda i: (0, 0)),
        ],
        out_specs=pl.BlockSpec((BLOCK, N), lambda i: (i, 0)),
        out_shape=jax.ShapeDtypeStruct((N, N), jnp.float32),
        compiler_params=pltpu.CompilerParams(
            dimension_semantics=("arbitrary",),
        ),
    )(Xn, Xn)
